# baseline (speedup 1.0000x reference)
import jax
import jax.numpy as jnp
from jax.experimental import pallas as pl
from jax.experimental.pallas import tpu as pltpu

NORM_PERCENT = 50


def _sumsq_body(x_ref, out_ref):
    xb = x_ref[...]
    out_ref[...] = jnp.sum(xb * xb, axis=1).reshape(1, 1, -1)


def _plan_body(k, s_ref, plan_ref):
    s = s_ref[0, :]
    n = s.shape[0]
    a = jax.lax.broadcast_in_dim(s, (n, n), (0,))
    b = jax.lax.broadcast_in_dim(s, (n, n), (1,))
    jidx = jax.lax.broadcasted_iota(jnp.int32, (n, n), 0)
    cidx = jax.lax.broadcasted_iota(jnp.int32, (n, n), 1)
    beats = (a > b) | ((a == b) & (jidx < cidx))
    rank = jnp.sum(beats.astype(jnp.int32), axis=0)
    keep = (rank < k).astype(jnp.int32)
    keep_j = jax.lax.broadcast_in_dim(keep, (n, n), (0,))
    cand = jnp.where((jidx <= cidx) & (keep_j == 1), jidx, 0)
    xblock = jnp.max(cand, axis=0)
    plan_ref[0, :] = xblock
    plan_ref[1, :] = keep


def _mul_body(plan_ref, x_ref, o_ref):
    i = pl.program_id(0)
    o_ref[...] = x_ref[...] * plan_ref[1, i].astype(jnp.float32)


def kernel(input):
    x = input
    _, C, H, W = x.shape
    k = int(float(NORM_PERCENT) / 100.0 * float(C))
    if k <= 0 or k >= C:
        k = C
    HW = H * W
    CB = 8
    nblk = C // CB
    LANES = 128
    SUB = HW // LANES

    x2 = x.reshape(C, HW)

    sumsq = pl.pallas_call(
        _sumsq_body,
        grid=(nblk,),
        in_specs=[pl.BlockSpec((CB, HW), lambda i: (i, 0))],
        out_specs=pl.BlockSpec((1, 1, CB), lambda i: (i, 0, 0)),
        out_shape=jax.ShapeDtypeStruct((nblk, 1, CB), jnp.float32),
    )(x2)

    plan = pl.pallas_call(
        lambda s_ref, plan_ref: _plan_body(k, s_ref, plan_ref),
        in_specs=[pl.BlockSpec((1, C), lambda: (0, 0))],
        out_specs=pl.BlockSpec((2, C), lambda: (0, 0)),
        out_shape=jax.ShapeDtypeStruct((2, C), jnp.int32),
    )(sumsq.reshape(1, C))

    x3 = x.reshape(C, SUB, LANES)
    grid_spec = pltpu.PrefetchScalarGridSpec(
        num_scalar_prefetch=1,
        grid=(C,),
        in_specs=[
            pl.BlockSpec((1, SUB, LANES), lambda i, pref: (pref[0, i], 0, 0)),
        ],
        out_specs=pl.BlockSpec((1, SUB, LANES), lambda i, pref: (i, 0, 0)),
    )
    out = pl.pallas_call(
        _mul_body,
        grid_spec=grid_spec,
        out_shape=jax.ShapeDtypeStruct((C, SUB, LANES), jnp.float32),
    )(plan, x3)

    return out.reshape(x.shape)


# speedup vs baseline: 1.9145x; 1.9145x over previous
import jax
import jax.numpy as jnp
from jax.experimental import pallas as pl
from jax.experimental.pallas import tpu as pltpu

NORM_PERCENT = 50
CB = 8


def _sumsq_body(x_ref, out_ref):
    xb = x_ref[...]
    out_ref[...] = jnp.sum(xb * xb, axis=1).reshape(1, 1, -1)


def _plan_body(k, s_ref, plan_ref):
    s = s_ref[0, :]
    n = s.shape[0]
    a = jax.lax.broadcast_in_dim(s, (n, n), (0,))
    b = jax.lax.broadcast_in_dim(s, (n, n), (1,))
    jidx = jax.lax.broadcasted_iota(jnp.int32, (n, n), 0)
    cidx = jax.lax.broadcasted_iota(jnp.int32, (n, n), 1)
    beats = (a > b) | ((a == b) & (jidx < cidx))
    rank = jnp.sum(beats.astype(jnp.int32), axis=0)
    plan_ref[0, :] = (rank < k).astype(jnp.int32)


def _mul_body(plan_ref, x_hbm, o_ref, xbuf, sems):
    b = pl.program_id(0)
    nb = pl.num_programs(0)
    C = plan_ref.shape[1]

    def issue(bb, slot):
        for ch in range(CB):
            cc = jnp.minimum(bb * CB + ch, C - 1)

            @pl.when((bb < nb) & (plan_ref[0, cc] == 1))
            def _():
                pltpu.make_async_copy(
                    x_hbm.at[pl.ds(cc, 1)],
                    xbuf.at[slot, pl.ds(ch, 1)],
                    sems.at[slot],
                ).start()

    @pl.when(b == 0)
    def _():
        issue(0, 0)

    issue(b + 1, (b + 1) % 2)

    slot = b % 2
    for ch in range(CB):
        c = b * CB + ch

        @pl.when(plan_ref[0, c] == 1)
        def _():
            pltpu.make_async_copy(
                x_hbm.at[pl.ds(c, 1)],
                xbuf.at[slot, pl.ds(ch, 1)],
                sems.at[slot],
            ).wait()
            o_ref[pl.ds(ch, 1), :] = xbuf[slot, pl.ds(ch, 1), :]

        @pl.when(plan_ref[0, c] == 0)
        def _():
            o_ref[pl.ds(ch, 1), :] = jnp.zeros_like(o_ref[pl.ds(ch, 1), :])


def kernel(input):
    x = input
    _, C, H, W = x.shape
    k = int(float(NORM_PERCENT) / 100.0 * float(C))
    if k <= 0 or k >= C:
        k = C
    HW = H * W
    nblk = C // CB

    x2 = x.reshape(C, HW)

    sumsq = pl.pallas_call(
        _sumsq_body,
        grid=(nblk,),
        in_specs=[pl.BlockSpec((CB, HW), lambda i: (i, 0))],
        out_specs=pl.BlockSpec((1, 1, CB), lambda i: (i, 0, 0)),
        out_shape=jax.ShapeDtypeStruct((nblk, 1, CB), jnp.float32),
    )(x2)

    plan = pl.pallas_call(
        lambda s_ref, plan_ref: _plan_body(k, s_ref, plan_ref),
        in_specs=[pl.BlockSpec((1, C), lambda: (0, 0))],
        out_specs=pl.BlockSpec((1, C), lambda: (0, 0)),
        out_shape=jax.ShapeDtypeStruct((1, C), jnp.int32),
    )(sumsq.reshape(1, C))

    grid_spec = pltpu.PrefetchScalarGridSpec(
        num_scalar_prefetch=1,
        grid=(nblk,),
        in_specs=[pl.BlockSpec(memory_space=pl.ANY)],
        out_specs=pl.BlockSpec((CB, HW), lambda i, pref: (i, 0)),
        scratch_shapes=[
            pltpu.VMEM((2, CB, HW), jnp.float32),
            pltpu.SemaphoreType.DMA((2,)),
        ],
    )
    out = pl.pallas_call(
        _mul_body,
        grid_spec=grid_spec,
        out_shape=jax.ShapeDtypeStruct((C, HW), jnp.float32),
    )(plan, x2)

    return out.reshape(x.shape)
